# Initial kernel scaffold; baseline (speedup 1.0000x reference)
import jax
import jax.numpy as jnp
from jax.experimental import pallas as pl


def kernel(pert_batch, W, b):
    raise NotImplementedError("write your pallas kernel here")


# speedup vs baseline: 17.1821x; 17.1821x over previous
import jax
import jax.numpy as jnp
from jax.experimental import pallas as pl

_B, _P, _D, _OUT = 4096, 32, 128, 128
_BLK = 256


def _fused_body(x_ref, w_ref, b_ref, o_ref):
    s = jnp.sum(x_ref[...], axis=1)
    y = jax.lax.dot_general(
        s, w_ref[...], (((1,), (1,)), ((), ())),
        preferred_element_type=jnp.float32,
        precision=jax.lax.Precision.HIGHEST,
    )
    o_ref[...] = y + b_ref[...]


def kernel(pert_batch, W, b):
    bscaled = (float(_P) * b).reshape(1, _OUT)
    grid = (_B // _BLK,)
    return pl.pallas_call(
        _fused_body,
        grid=grid,
        in_specs=[
            pl.BlockSpec((_BLK, _P, _D), lambda i: (i, 0, 0)),
            pl.BlockSpec((_OUT, _D), lambda i: (0, 0)),
            pl.BlockSpec((1, _OUT), lambda i: (0, 0)),
        ],
        out_specs=pl.BlockSpec((_BLK, _OUT), lambda i: (i, 0)),
        out_shape=jax.ShapeDtypeStruct((_B, _OUT), jnp.float32),
    )(pert_batch, W, bscaled)
